# baseline (speedup 1.0000x reference)
import numpy as np
import jax
import jax.numpy as jnp
from jax import lax
from jax.experimental import pallas as pl
from jax.experimental.pallas import tpu as pltpu
from jax.experimental.pallas import tpu_sc as plsc

_A = 1000
_D = 513
_S = 0.13
_IMG = 512
_CROP = 362
_B = 4
_TOP = (_IMG - _CROP) // 2
_E = 516
_AW = _B * _E
_RW = 12
_NJ = 23
_JP = _NJ * 16
_KCH = 10
_NCH = _A // _KCH
_CHW = _KCH * _AW
_PP = 1024
_SCALE = float(np.pi / _A)


def _params_host():
    th = np.linspace(0.0, np.pi, _A + 1)[:-1]
    delta = 2.0 * _S / (_D - 1)
    dxx = 2.0 * _S / (_IMG - 1)
    u = dxx * np.cos(th) / delta
    v = dxx * np.sin(th) / delta
    xs0 = -_S + _TOP * dxx
    c = (xs0 * (np.cos(th) + np.sin(th)) + _S) / delta + 1.0
    grp_q = np.abs(v) >= np.abs(u)
    order = np.concatenate([np.nonzero(grp_q)[0], np.nonzero(~grp_q)[0]])
    assert order.shape[0] == _A and int(grp_q.sum()) == _A // 2
    outer = np.where(grp_q, u, v)[order]
    lane = np.where(grp_q, v, u)[order]
    p = np.zeros((3, _PP), np.float32)
    p[0, :_A] = c[order]
    p[1, :_A] = outer
    p[2, :_A] = lane
    return p.reshape(-1), order


_PARAMS, _ORDER = _params_host()


def _body(tbl, par, out, parv, tb0, tb1, acc, sem0, sem1):
    wid = lax.axis_index("s") * 2 + lax.axis_index("c")
    i0 = jnp.minimum(wid * _RW, _CROP - _RW)
    i0f = i0.astype(jnp.float32)
    pltpu.sync_copy(par, parv)

    zero = jnp.zeros((16,), jnp.float32)

    def _zero(il, carry):
        for b in range(2 * _B):
            for jt in range(_NJ):
                acc[pl.ds((b * _RW + il) * _JP + jt * 16, 16)] = zero
        return carry

    lax.fori_loop(0, _RW, _zero, 0)

    qf = lax.iota(jnp.int32, 16).astype(jnp.float32)

    def _do_chunk(ch, tb):
        abase = ch * _KCH
        accoff = jnp.where(ch < _NCH // 2, 0, _B * _RW * _JP)

        def _angle(k, carry):
            av = lax.broadcast(abase + k, (16,))
            cv = plsc.load_gather(parv, [av])
            uv = plsc.load_gather(parv, [av + _PP])
            vv = plsc.load_gather(parv, [av + 2 * _PP])
            kbase = k * _AW
            rb0 = cv + i0f * uv

            @plsc.parallel_loop(0, _RW, unroll=4)
            def _row(il):
                rb = rb0 + il.astype(jnp.float32) * uv
                arow = accoff + il * _JP
                for jt in range(_NJ):
                    pos = rb + vv * (qf + (jt * 16.0))
                    gi = pos.astype(jnp.int32)
                    f1 = pos - gi.astype(jnp.float32)
                    w0 = 1.0 - f1
                    g0 = jnp.clip(gi, 0, _E - 2) + kbase
                    for b in range(_B):
                        gb = g0 + (b * _E)
                        t0 = plsc.load_gather(tb, [gb])
                        t1 = plsc.load_gather(tb, [gb + 1])
                        plsc.addupdate(
                            acc.at[pl.ds(arow + (b * _RW * _JP + jt * 16), 16)],
                            w0 * t0 + f1 * t1,
                        )

            return carry

        lax.fori_loop(0, _KCH, _angle, 0)

    def _start(ch, buf, sem):
        pltpu.make_async_copy(tbl.at[pl.ds(ch * _CHW, _CHW)], buf, sem).start()

    def _wait(buf, sem):
        pltpu.make_async_copy(tbl.at[pl.ds(0, _CHW)], buf, sem).wait()

    _start(0, tb0, sem0)

    def _outer(c, carry):
        _start(2 * c + 1, tb1, sem1)
        _wait(tb0, sem0)
        _do_chunk(2 * c, tb0)
        _start(lax.rem(2 * c + 2, _NCH), tb0, sem0)
        _wait(tb1, sem1)
        _do_chunk(2 * c + 1, tb1)
        return carry

    lax.fori_loop(0, _NCH // 2, _outer, 0)
    _wait(tb0, sem0)

    sc = jnp.float32(_SCALE)

    def _fin(il, carry):
        for b in range(2 * _B):
            for jt in range(_NJ):
                off = (b * _RW + il) * _JP + jt * 16
                acc[pl.ds(off, 16)] = acc[pl.ds(off, 16)] * sc
        return carry

    lax.fori_loop(0, _RW, _fin, 0)

    for b in range(2 * _B):
        pltpu.sync_copy(
            acc.at[pl.ds(b * _RW * _JP, _RW * _JP)],
            out.at[pl.ds((b * _CROP + i0) * _JP, _RW * _JP)],
        )


def kernel(x):
    t = jnp.transpose(x[:, 0, :, :], (1, 0, 2))
    t = t[jnp.asarray(_ORDER)]
    t = jnp.pad(t, ((0, 0), (0, 0), (1, 2)))
    tbl = t.reshape(-1)
    par = jnp.asarray(_PARAMS)
    mesh = plsc.VectorSubcoreMesh(core_axis_name="c", subcore_axis_name="s")
    kfn = pl.kernel(
        _body,
        out_type=jax.ShapeDtypeStruct((2 * _B * _CROP * _JP,), jnp.float32),
        mesh=mesh,
        scratch_types=[
            pltpu.VMEM((3 * _PP,), jnp.float32),
            pltpu.VMEM((_CHW,), jnp.float32),
            pltpu.VMEM((_CHW,), jnp.float32),
            pltpu.VMEM((2 * _B * _RW * _JP,), jnp.float32),
            pltpu.SemaphoreType.DMA,
            pltpu.SemaphoreType.DMA,
        ],
        compiler_params=pltpu.CompilerParams(needs_layout_passes=False),
    )
    outp = kfn(tbl, par).reshape(2, _B, 1, _CROP, _JP)[:, :, :, :, :_CROP]
    return outp[0] + jnp.transpose(outp[1], (0, 1, 3, 2))


# speedup vs baseline: 4.4382x; 4.4382x over previous
import numpy as np
import jax
import jax.numpy as jnp
from jax import lax
from jax.experimental import pallas as pl
from jax.experimental.pallas import tpu as pltpu
from jax.experimental.pallas import tpu_sc as plsc

_A = 1000
_D = 513
_S = 0.13
_IMG = 512
_CROP = 362
_B = 4
_TOP = (_IMG - _CROP) // 2
_E = 516
_AW = _B * _E
_RW = 12
_NJ = 23
_JP = _NJ * 16
_KCH = 10
_NCH = _A // _KCH
_CHW = _KCH * _AW
_PP = 1024
_SCALE = float(np.pi / _A)


def _params_host():
    th = np.linspace(0.0, np.pi, _A + 1)[:-1]
    delta = 2.0 * _S / (_D - 1)
    dxx = 2.0 * _S / (_IMG - 1)
    u = dxx * np.cos(th) / delta
    v = dxx * np.sin(th) / delta
    xs0 = -_S + _TOP * dxx
    c = (xs0 * (np.cos(th) + np.sin(th)) + _S) / delta + 1.0
    grp_q = np.abs(v) >= np.abs(u)
    order = np.concatenate([np.nonzero(grp_q)[0], np.nonzero(~grp_q)[0]])
    assert order.shape[0] == _A and int(grp_q.sum()) == _A // 2
    outer = np.where(grp_q, u, v)[order]
    lane = np.where(grp_q, v, u)[order]
    p = np.zeros((3, _PP), np.float32)
    p[0, :_A] = c[order]
    p[1, :_A] = outer
    p[2, :_A] = lane
    return p.reshape(-1), order


_PARAMS, _ORDER = _params_host()


def _body(tbl, par, out, parv, tb0, tb1, acc, sem0, sem1):
    wid = lax.axis_index("s") * 2 + lax.axis_index("c")
    i0 = jnp.minimum(wid * _RW, _CROP - _RW)
    i0f = i0.astype(jnp.float32)
    pltpu.sync_copy(par, parv)

    zero = jnp.zeros((16,), jnp.float32)

    def _zero(il, carry):
        for b in range(2 * _B):
            for jt in range(_NJ):
                acc[pl.ds((b * _RW + il) * _JP + jt * 16, 16)] = zero
        return carry

    lax.fori_loop(0, _RW, _zero, 0)

    qf = lax.iota(jnp.int32, 16).astype(jnp.float32)

    def _do_chunk(ch, tb):
        abase = ch * _KCH
        accoff = jnp.where(ch < _NCH // 2, 0, _B * _RW * _JP)

        def _angle(k, carry):
            av = lax.broadcast(abase + k, (16,))
            cv = plsc.load_gather(parv, [av])
            uv = plsc.load_gather(parv, [av + _PP])
            vv = plsc.load_gather(parv, [av + 2 * _PP])
            kbase = k * _AW
            rb0 = cv + i0f * uv

            @plsc.parallel_loop(0, _RW, unroll=1)
            def _row(il):
                rb = rb0 + il.astype(jnp.float32) * uv
                arow = accoff + il * _JP

                @plsc.parallel_loop(0, _NJ, unroll=2)
                def _jvec(jt):
                    pos = rb + vv * (qf + jt.astype(jnp.float32) * 16.0)
                    gi = pos.astype(jnp.int32)
                    f1 = pos - gi.astype(jnp.float32)
                    w0 = 1.0 - f1
                    g0 = jnp.clip(gi, 0, _E - 2) + kbase
                    for b in range(_B):
                        gb = g0 + (b * _E)
                        t0 = plsc.load_gather(tb, [gb])
                        t1 = plsc.load_gather(tb, [gb + 1])
                        plsc.addupdate(
                            acc.at[pl.ds(arow + b * _RW * _JP + jt * 16, 16)],
                            w0 * t0 + f1 * t1,
                        )

            return carry

        lax.fori_loop(0, _KCH, _angle, 0)

    def _start(ch, buf, sem):
        pltpu.make_async_copy(tbl.at[pl.ds(ch * _CHW, _CHW)], buf, sem).start()

    def _wait(buf, sem):
        pltpu.make_async_copy(tbl.at[pl.ds(0, _CHW)], buf, sem).wait()

    _start(0, tb0, sem0)

    def _outer(c, carry):
        _start(2 * c + 1, tb1, sem1)
        _wait(tb0, sem0)
        _do_chunk(2 * c, tb0)
        _start(lax.rem(2 * c + 2, _NCH), tb0, sem0)
        _wait(tb1, sem1)
        _do_chunk(2 * c + 1, tb1)
        return carry

    lax.fori_loop(0, _NCH // 2, _outer, 0)
    _wait(tb0, sem0)

    sc = jnp.float32(_SCALE)

    def _fin(il, carry):
        for b in range(2 * _B):
            for jt in range(_NJ):
                off = (b * _RW + il) * _JP + jt * 16
                acc[pl.ds(off, 16)] = acc[pl.ds(off, 16)] * sc
        return carry

    lax.fori_loop(0, _RW, _fin, 0)

    for b in range(2 * _B):
        pltpu.sync_copy(
            acc.at[pl.ds(b * _RW * _JP, _RW * _JP)],
            out.at[pl.ds((b * _CROP + i0) * _JP, _RW * _JP)],
        )


def kernel(x):
    t = jnp.transpose(x[:, 0, :, :], (1, 0, 2))
    t = t[jnp.asarray(_ORDER)]
    t = jnp.pad(t, ((0, 0), (0, 0), (1, 2)))
    tbl = t.reshape(-1)
    par = jnp.asarray(_PARAMS)
    mesh = plsc.VectorSubcoreMesh(core_axis_name="c", subcore_axis_name="s")
    kfn = pl.kernel(
        _body,
        out_type=jax.ShapeDtypeStruct((2 * _B * _CROP * _JP,), jnp.float32),
        mesh=mesh,
        scratch_types=[
            pltpu.VMEM((3 * _PP,), jnp.float32),
            pltpu.VMEM((_CHW,), jnp.float32),
            pltpu.VMEM((_CHW,), jnp.float32),
            pltpu.VMEM((2 * _B * _RW * _JP,), jnp.float32),
            pltpu.SemaphoreType.DMA,
            pltpu.SemaphoreType.DMA,
        ],
        compiler_params=pltpu.CompilerParams(needs_layout_passes=False),
    )
    outp = kfn(tbl, par).reshape(2, _B, 1, _CROP, _JP)[:, :, :, :, :_CROP]
    return outp[0] + jnp.transpose(outp[1], (0, 1, 3, 2))
